# Initial kernel scaffold; baseline (speedup 1.0000x reference)
import jax
import jax.numpy as jnp
from jax.experimental import pallas as pl


def kernel(inputs, params):
    raise NotImplementedError("write your pallas kernel here")


# speedup vs baseline: 1.4364x; 1.4364x over previous
import functools

import jax
import jax.numpy as jnp
from jax import lax
from jax.experimental import pallas as pl
from jax.experimental.pallas import tpu as pltpu
from jax.experimental.pallas import tpu_sc as plsc

NC = 2
NS = 16
NW = NC * NS

BATCH = 16384
FIELDS = 26
EMBED_DIM = 32
TOTAL = BATCH * FIELDS
B_PER_W = TOTAL // NW
CHUNK = 128
NCH = B_PER_W // CHUNK

_mesh = plsc.VectorSubcoreMesh(core_axis_name="c", subcore_axis_name="s")


@functools.partial(
    pl.kernel,
    mesh=_mesh,
    out_type=jax.ShapeDtypeStruct((TOTAL, EMBED_DIM), jnp.float32),
    scratch_types=[
        pltpu.VMEM((NCH, CHUNK), jnp.int32),
        pltpu.VMEM((CHUNK, EMBED_DIM), jnp.float32),
        pltpu.SemaphoreType.DMA,
    ],
    compiler_params=pltpu.CompilerParams(use_tc_tiling_on_sc=False),
)
def _gather_kernel(idx_hbm, table_hbm, out_hbm, idx_v, rows_v, sem):
    wid = lax.axis_index("s") * NC + lax.axis_index("c")
    base = wid * B_PER_W
    pltpu.sync_copy(idx_hbm.at[wid], idx_v)

    def chunk_body(j, carry):
        pltpu.async_copy(table_hbm.at[idx_v.at[j]], rows_v, sem).wait()
        pltpu.sync_copy(rows_v, out_hbm.at[pl.ds(base + j * CHUNK, CHUNK)])
        return carry

    lax.fori_loop(0, NCH, chunk_body, 0)


def kernel(inputs, params):
    idx = inputs.reshape(NW, NCH, CHUNK).astype(jnp.int32)
    out = _gather_kernel(idx, params)
    return out.reshape(BATCH, FIELDS, EMBED_DIM)


# speedup vs baseline: 1.5741x; 1.0958x over previous
import functools

import jax
import jax.numpy as jnp
from jax import lax
from jax.experimental import pallas as pl
from jax.experimental.pallas import tpu as pltpu
from jax.experimental.pallas import tpu_sc as plsc

NC = 2
NS = 16
NW = NC * NS

BATCH = 16384
FIELDS = 26
EMBED_DIM = 32
TOTAL = BATCH * FIELDS
B_PER_W = TOTAL // NW
CHUNK = 128
NCH = B_PER_W // CHUNK

_mesh = plsc.VectorSubcoreMesh(core_axis_name="c", subcore_axis_name="s")


K = 8
NG = NCH // K


@functools.partial(
    pl.kernel,
    mesh=_mesh,
    out_type=jax.ShapeDtypeStruct((TOTAL, EMBED_DIM), jnp.float32),
    scratch_types=[
        pltpu.VMEM((NCH, CHUNK), jnp.int32),
        pltpu.VMEM((K, CHUNK, EMBED_DIM), jnp.float32),
    ]
    + [pltpu.SemaphoreType.DMA] * (2 * K),
    compiler_params=pltpu.CompilerParams(use_tc_tiling_on_sc=False),
)
def _gather_kernel(idx_hbm, table_hbm, out_hbm, idx_v, rows_v, *sems):
    gsem = sems[:K]
    osem = sems[K:]
    wid = lax.axis_index("s") * NC + lax.axis_index("c")
    base = wid * B_PER_W
    pltpu.sync_copy(idx_hbm.at[wid], idx_v)

    def group_body(i, carry):
        for s in range(K):
            c = i * K + s

            @pl.when(i > 0)
            def _():
                pltpu.make_async_copy(
                    rows_v.at[s],
                    out_hbm.at[pl.ds(base + c * CHUNK, CHUNK)],
                    osem[s],
                ).wait()

            pltpu.async_copy(table_hbm.at[idx_v.at[c]], rows_v.at[s], gsem[s])
        for s in range(K):
            c = i * K + s
            pltpu.make_async_copy(
                table_hbm.at[idx_v.at[c]], rows_v.at[s], gsem[s]
            ).wait()
            pltpu.async_copy(
                rows_v.at[s],
                out_hbm.at[pl.ds(base + c * CHUNK, CHUNK)],
                osem[s],
            )
        return carry

    lax.fori_loop(0, NG, group_body, 0)
    for s in range(K):
        c = (NG - 1) * K + s
        pltpu.make_async_copy(
            rows_v.at[s],
            out_hbm.at[pl.ds(base + c * CHUNK, CHUNK)],
            osem[s],
        ).wait()


def kernel(inputs, params):
    idx = inputs.reshape(NW, NCH, CHUNK).astype(jnp.int32)
    out = _gather_kernel(idx, params)
    return out.reshape(BATCH, FIELDS, EMBED_DIM)
